# Initial kernel scaffold; baseline (speedup 1.0000x reference)
import functools

import jax
import jax.numpy as jnp
from jax import lax
from jax.experimental import pallas as pl
from jax.experimental.pallas import tpu as pltpu

N_OBJ = 1000
N_REL = 5000
NUM_OBJ_CLS = 151
NUM_REL_CLS = 51
OBJ_DIM = 4096
REL_DIM = 4096
HID = 512
T_STEPS = 3
NMS_THRESH = 0.3

NP_OBJ = 1024
CP = 256
NP_REL = 5120
RB = 512
NEG = -1e30


def _probs_body(logits_ref, out_ref):
    x = logits_ref[...]
    m = jnp.max(x, axis=1, keepdims=True)
    e = jnp.exp(x - m)
    out_ref[...] = e / jnp.sum(e, axis=1, keepdims=True)


def _probs_call(logits_p):
    return pl.pallas_call(
        _probs_body,
        out_shape=jax.ShapeDtypeStruct((N_OBJ, CP), jnp.float32),
    )(logits_p)


def _nms_body(x1r, y1r, x2r, y2r, sr, x1c, y1c, x2c, y2c, sc, keep_ref):
    x1_r = x1r[0]
    y1_r = y1r[0]
    x2_r = x2r[0]
    y2_r = y2r[0]
    s_r = sr[0]
    x1_c = x1c[0]
    y1_c = y1c[0]
    x2_c = x2c[0]
    y2_c = y2c[0]
    s_c = sc[0]

    area_r = (x2_r - x1_r) * (y2_r - y1_r)
    area_c = (x2_c - x1_c) * (y2_c - y1_c)
    xx1 = jnp.maximum(x1_c, x1_r)
    yy1 = jnp.maximum(y1_c, y1_r)
    xx2 = jnp.minimum(x2_c, x2_r)
    yy2 = jnp.minimum(y2_c, y2_r)
    w = jnp.maximum(xx2 - xx1, 0.0)
    h = jnp.maximum(yy2 - yy1, 0.0)
    inter = w * h
    iou = inter / (area_c + area_r - inter + 1e-8)

    ii = lax.broadcasted_iota(jnp.int32, (NP_OBJ, NP_OBJ), 0)
    jj = lax.broadcasted_iota(jnp.int32, (NP_OBJ, NP_OBJ), 1)
    prec = (s_r > s_c) | ((s_r == s_c) & (jj < ii))
    overlap = (iou > NMS_THRESH) & prec & (jj < N_OBJ) & (ii < N_OBJ)
    q = jnp.transpose(overlap).astype(jnp.float32)

    lane = lax.broadcasted_iota(jnp.int32, (1, NP_OBJ), 1)
    valid = (lane < N_OBJ).astype(jnp.float32)

    def cond(state):
        _, _, und = state
        return jnp.sum(und) > 0.0

    def body(state):
        kept, supp, und = state
        blocked = jnp.dot(kept, q, preferred_element_type=jnp.float32)
        new_supp = und * (blocked > 0.0).astype(jnp.float32)
        supp = supp + new_supp
        und = und - new_supp
        notsupp = valid - supp
        pending = jnp.dot(notsupp, q, preferred_element_type=jnp.float32)
        new_kept = und * (pending == 0.0).astype(jnp.float32)
        kept = kept + new_kept
        und = und - new_kept
        return kept, supp, und

    z = jnp.zeros((1, NP_OBJ), jnp.float32)
    kept, _, _ = lax.while_loop(cond, body, (z, 1.0 - valid, valid))
    keep_ref[...] = kept[None]


def _nms_call(rows, cols):
    nclass = NUM_OBJ_CLS - 1
    row_spec = pl.BlockSpec((1, 1, NP_OBJ), lambda c: (c, 0, 0))
    col_spec = pl.BlockSpec((1, NP_OBJ, 1), lambda c: (c, 0, 0))
    return pl.pallas_call(
        _nms_body,
        grid=(nclass,),
        in_specs=[row_spec] * 5 + [col_spec] * 5,
        out_specs=pl.BlockSpec((1, 1, NP_OBJ), lambda c: (c, 0, 0)),
        out_shape=jax.ShapeDtypeStruct((nclass, 1, NP_OBJ), jnp.float32),
    )(*rows, *cols)


def _relmm_body(x_ref, w_ref, b_ref, o_ref):
    o_ref[...] = (
        jnp.dot(x_ref[...], w_ref[...], preferred_element_type=jnp.float32)
        + b_ref[...]
    )


def _relmm_call(vr, w, b):
    mb = 200
    return pl.pallas_call(
        _relmm_body,
        grid=(N_REL // mb,),
        in_specs=[
            pl.BlockSpec((mb, REL_DIM), lambda i: (i, 0)),
            pl.BlockSpec((REL_DIM, HID), lambda i: (0, 0)),
            pl.BlockSpec((1, HID), lambda i: (0, 0)),
        ],
        out_specs=pl.BlockSpec((mb, HID), lambda i: (i, 0)),
        out_shape=jax.ShapeDtypeStruct((N_REL, HID), jnp.float32),
    )(vr, w, b)


def _objmm_body(f_ref, w_ref, b_ref, lg_ref, mask_ref, probs_ref, wemb_ref,
                h_ref, pred_ref):
    lg = lg_ref[...]
    mask = mask_ref[...]
    lane = lax.broadcasted_iota(jnp.int32, lg.shape, 1)
    lp = mask * lg + (1.0 - mask) * (-1000.0)
    lp = jnp.where(lane < NUM_OBJ_CLS, lp, NEG)
    m = jnp.max(lp, axis=1, keepdims=True)
    e = jnp.exp(lp - m)
    p2 = e / jnp.sum(e, axis=1, keepdims=True)
    emb = jnp.dot(p2, wemb_ref[...], preferred_element_type=jnp.float32)
    h = jnp.dot(f_ref[...], w_ref[...], preferred_element_type=jnp.float32)
    h_ref[...] = jnp.tanh(h + b_ref[...] + emb)

    mp = mask * probs_ref[...]
    adj = jnp.where((lane >= 1) & (lane < NUM_OBJ_CLS), mp, -1.0)
    mx = jnp.max(adj, axis=1, keepdims=True)
    cand = jnp.where(adj == mx, lane, NUM_OBJ_CLS + 1)
    pred_ref[...] = jnp.min(cand, axis=1, keepdims=True)


def _objmm_call(fmaps, w, b, logits_p, mask_p, probs, wemb_p):
    mb = 200
    return pl.pallas_call(
        _objmm_body,
        grid=(N_OBJ // mb,),
        in_specs=[
            pl.BlockSpec((mb, OBJ_DIM), lambda i: (i, 0)),
            pl.BlockSpec((OBJ_DIM, HID), lambda i: (0, 0)),
            pl.BlockSpec((1, HID), lambda i: (0, 0)),
            pl.BlockSpec((mb, CP), lambda i: (i, 0)),
            pl.BlockSpec((mb, CP), lambda i: (i, 0)),
            pl.BlockSpec((mb, CP), lambda i: (i, 0)),
            pl.BlockSpec((CP, HID), lambda i: (0, 0)),
        ],
        out_specs=[
            pl.BlockSpec((mb, HID), lambda i: (i, 0)),
            pl.BlockSpec((mb, 1), lambda i: (i, 0)),
        ],
        out_shape=[
            jax.ShapeDtypeStruct((N_OBJ, HID), jnp.float32),
            jax.ShapeDtypeStruct((N_OBJ, 1), jnp.int32),
        ],
    )(fmaps, w, b, logits_p, mask_p, probs, wemb_p)


def _ggnn_body(vr0_ref, objh0_ref, subr_ref, obr_ref, subc_ref, obc_ref,
               wmsg_ref, wout_ref, bout_ref, out_ref, vr_ref, objh_ref):
    nblk = NP_REL // RB
    vr_ref[...] = vr0_ref[...]
    objh_ref[...] = objh0_ref[...]

    iota_obj_row = lax.broadcasted_iota(jnp.int32, (NP_OBJ, RB), 0)
    iota_obj_lane = lax.broadcasted_iota(jnp.int32, (RB, NP_OBJ), 1)

    for _ in range(T_STEPS):
        msg = jnp.zeros((NP_OBJ, HID), jnp.float32)
        for rb in range(nblk):
            sub_blk = subr_ref[:, rb * RB:(rb + 1) * RB]
            ob_blk = obr_ref[:, rb * RB:(rb + 1) * RB]
            oh = ((sub_blk == iota_obj_row).astype(jnp.float32)
                  + (ob_blk == iota_obj_row).astype(jnp.float32))
            vr_blk = vr_ref[rb * RB:(rb + 1) * RB, :]
            msg = msg + jnp.dot(oh, vr_blk, preferred_element_type=jnp.float32)
        nrm = jnp.sqrt(jnp.sum(msg * msg, axis=1, keepdims=True))
        msg = msg / (nrm + 1e-8)
        objh_ref[...] = jnp.tanh(
            objh_ref[...]
            + jnp.dot(msg, wmsg_ref[...], preferred_element_type=jnp.float32))
        obj_h = objh_ref[...]
        for rb in range(nblk):
            sub_blk = subc_ref[rb * RB:(rb + 1) * RB, :]
            ob_blk = obc_ref[rb * RB:(rb + 1) * RB, :]
            g = ((sub_blk == iota_obj_lane).astype(jnp.float32)
                 + (ob_blk == iota_obj_lane).astype(jnp.float32))
            gsum = jnp.dot(g, obj_h, preferred_element_type=jnp.float32)
            vr_ref[rb * RB:(rb + 1) * RB, :] = jnp.tanh(
                vr_ref[rb * RB:(rb + 1) * RB, :] + gsum)

    obj_h = objh_ref[...]
    w1 = wout_ref[0:HID, :]
    w2 = wout_ref[HID:2 * HID, :]
    w3 = wout_ref[2 * HID:3 * HID, :]
    a1 = jnp.dot(obj_h, w1, preferred_element_type=jnp.float32)
    a2 = jnp.dot(obj_h, w2, preferred_element_type=jnp.float32)
    for rb in range(nblk):
        sub_blk = subc_ref[rb * RB:(rb + 1) * RB, :]
        ob_blk = obc_ref[rb * RB:(rb + 1) * RB, :]
        gs = (sub_blk == iota_obj_lane).astype(jnp.float32)
        go = (ob_blk == iota_obj_lane).astype(jnp.float32)
        out_ref[rb * RB:(rb + 1) * RB, :] = (
            jnp.dot(gs, a1, preferred_element_type=jnp.float32)
            + jnp.dot(go, a2, preferred_element_type=jnp.float32)
            + jnp.dot(vr_ref[rb * RB:(rb + 1) * RB, :], w3,
                      preferred_element_type=jnp.float32)
            + bout_ref[...])


def _ggnn_call(vr0_p, objh0_p, sub_r, ob_r, sub_c, ob_c, wmsg, wout_p, bout_p):
    return pl.pallas_call(
        _ggnn_body,
        scratch_shapes=[
            pltpu.VMEM((NP_REL, HID), jnp.float32),
            pltpu.VMEM((NP_OBJ, HID), jnp.float32),
        ],
        out_shape=jax.ShapeDtypeStruct((NP_REL, 128), jnp.float32),
    )(vr0_p, objh0_p, sub_r, ob_r, sub_c, ob_c, wmsg, wout_p, bout_p)


def kernel(im_inds, obj_fmaps, obj_logits, rel_inds, vr, boxes_per_cls,
           W_obj, b_obj, W_rel, b_rel, W_emb, W_msg, W_out, b_out):
    f32 = jnp.float32
    nclass = NUM_OBJ_CLS - 1

    logits_p = jnp.pad(obj_logits, ((0, 0), (0, CP - NUM_OBJ_CLS)),
                       constant_values=NEG)
    probs = _probs_call(logits_p)

    bt = jnp.transpose(boxes_per_cls[:, 1:, :], (1, 2, 0))
    bt = jnp.pad(bt, ((0, 0), (0, 0), (0, NP_OBJ - N_OBJ)))
    st = jnp.transpose(probs[:, 1:NUM_OBJ_CLS])
    st = jnp.pad(st, ((0, 0), (0, NP_OBJ - N_OBJ)), constant_values=-1.0)
    rows = [bt[:, 0].reshape(nclass, 1, NP_OBJ),
            bt[:, 1].reshape(nclass, 1, NP_OBJ),
            bt[:, 2].reshape(nclass, 1, NP_OBJ),
            bt[:, 3].reshape(nclass, 1, NP_OBJ),
            st.reshape(nclass, 1, NP_OBJ)]
    cols = [bt[:, 0].reshape(nclass, NP_OBJ, 1),
            bt[:, 1].reshape(nclass, NP_OBJ, 1),
            bt[:, 2].reshape(nclass, NP_OBJ, 1),
            bt[:, 3].reshape(nclass, NP_OBJ, 1),
            st.reshape(nclass, NP_OBJ, 1)]
    keep = _nms_call(rows, cols)
    keep2 = jnp.transpose(keep[:, 0, :N_OBJ])
    mask = jnp.concatenate([jnp.zeros((N_OBJ, 1), f32), keep2], axis=1)
    mask_p = jnp.pad(mask, ((0, 0), (0, CP - NUM_OBJ_CLS)))

    vr_h0 = _relmm_call(vr, W_rel, b_rel.reshape(1, HID))
    wemb_p = jnp.pad(W_emb, ((0, CP - NUM_OBJ_CLS), (0, 0)))
    obj_h0, preds = _objmm_call(obj_fmaps, W_obj, b_obj.reshape(1, HID),
                                logits_p, mask_p, probs, wemb_p)

    sub = rel_inds[:, 1]
    ob = rel_inds[:, 2]
    sub_p = jnp.pad(sub, (0, NP_REL - N_REL), constant_values=-1)
    ob_p = jnp.pad(ob, (0, NP_REL - N_REL), constant_values=-1)
    vr0_p = jnp.pad(vr_h0, ((0, NP_REL - N_REL), (0, 0)))
    objh0_p = jnp.pad(obj_h0, ((0, NP_OBJ - N_OBJ), (0, 0)))
    wout_p = jnp.pad(W_out, ((0, 0), (0, 128 - NUM_REL_CLS)))
    bout_p = jnp.pad(b_out, (0, 128 - NUM_REL_CLS)).reshape(1, 128)
    rel_p = _ggnn_call(vr0_p, objh0_p,
                       sub_p.reshape(1, NP_REL), ob_p.reshape(1, NP_REL),
                       sub_p.reshape(NP_REL, 1), ob_p.reshape(NP_REL, 1),
                       W_msg, wout_p, bout_p)
    rel_logits = rel_p[:N_REL, :NUM_REL_CLS]

    obj_preds = preds.reshape(N_OBJ).astype(jnp.int32)
    return (obj_logits, obj_preds, rel_logits)


# speedup vs baseline: 6.2472x; 6.2472x over previous
import functools

import jax
import jax.numpy as jnp
from jax import lax
from jax.experimental import pallas as pl
from jax.experimental.pallas import tpu as pltpu

N_OBJ = 1000
N_REL = 5000
NUM_OBJ_CLS = 151
NUM_REL_CLS = 51
OBJ_DIM = 4096
REL_DIM = 4096
HID = 512
T_STEPS = 3
NMS_THRESH = 0.3

NP_OBJ = 1024
CP = 256
NP_REL = 5120
RB = 512
NEG = -1e30


def _probs_body(logits_ref, out_ref):
    x = logits_ref[...]
    m = jnp.max(x, axis=1, keepdims=True)
    e = jnp.exp(x - m)
    out_ref[...] = e / jnp.sum(e, axis=1, keepdims=True)


def _probs_call(logits_p):
    return pl.pallas_call(
        _probs_body,
        out_shape=jax.ShapeDtypeStruct((N_OBJ, CP), jnp.float32),
    )(logits_p)


def _nms_body(x1r, y1r, x2r, y2r, sr, x1c, y1c, x2c, y2c, sc, keep_ref):
    x1_r = x1r[0]
    y1_r = y1r[0]
    x2_r = x2r[0]
    y2_r = y2r[0]
    s_r = sr[0]
    x1_c = x1c[0]
    y1_c = y1c[0]
    x2_c = x2c[0]
    y2_c = y2c[0]
    s_c = sc[0]

    area_r = (x2_r - x1_r) * (y2_r - y1_r)
    area_c = (x2_c - x1_c) * (y2_c - y1_c)
    xx1 = jnp.maximum(x1_c, x1_r)
    yy1 = jnp.maximum(y1_c, y1_r)
    xx2 = jnp.minimum(x2_c, x2_r)
    yy2 = jnp.minimum(y2_c, y2_r)
    w = jnp.maximum(xx2 - xx1, 0.0)
    h = jnp.maximum(yy2 - yy1, 0.0)
    inter = w * h
    iou = inter / (area_c + area_r - inter + 1e-8)

    ii = lax.broadcasted_iota(jnp.int32, (NP_OBJ, NP_OBJ), 0)
    jj = lax.broadcasted_iota(jnp.int32, (NP_OBJ, NP_OBJ), 1)
    prec = (s_c > s_r) | ((s_c == s_r) & (ii < jj))
    q = ((iou > NMS_THRESH) & prec & (ii < N_OBJ) & (jj < N_OBJ)
         ).astype(jnp.float32)

    lane = lax.broadcasted_iota(jnp.int32, (1, NP_OBJ), 1)
    valid = (lane < N_OBJ).astype(jnp.float32)

    def cond(state):
        _, _, und = state
        return jnp.sum(und) > 0.0

    def body(state):
        kept, supp, und = state
        blocked = jnp.dot(kept, q, preferred_element_type=jnp.float32)
        new_supp = und * (blocked > 0.0).astype(jnp.float32)
        supp = supp + new_supp
        und = und - new_supp
        notsupp = valid - supp
        pending = jnp.dot(notsupp, q, preferred_element_type=jnp.float32)
        new_kept = und * (pending == 0.0).astype(jnp.float32)
        kept = kept + new_kept
        und = und - new_kept
        return kept, supp, und

    z = jnp.zeros((1, NP_OBJ), jnp.float32)
    kept, _, _ = lax.while_loop(cond, body, (z, 1.0 - valid, valid))
    keep_ref[...] = kept[None]


def _nms_call(rows, cols):
    nclass = NUM_OBJ_CLS - 1
    row_spec = pl.BlockSpec((1, 1, NP_OBJ), lambda c: (c, 0, 0))
    col_spec = pl.BlockSpec((1, NP_OBJ, 1), lambda c: (c, 0, 0))
    return pl.pallas_call(
        _nms_body,
        grid=(nclass,),
        in_specs=[row_spec] * 5 + [col_spec] * 5,
        out_specs=pl.BlockSpec((1, 1, NP_OBJ), lambda c: (c, 0, 0)),
        out_shape=jax.ShapeDtypeStruct((nclass, 1, NP_OBJ), jnp.float32),
    )(*rows, *cols)


def _relmm_body(x_ref, w_ref, b_ref, o_ref):
    o_ref[...] = (
        jnp.dot(x_ref[...], w_ref[...], preferred_element_type=jnp.float32)
        + b_ref[...]
    )


def _relmm_call(vr, w, b):
    mb = 200
    return pl.pallas_call(
        _relmm_body,
        grid=(N_REL // mb,),
        in_specs=[
            pl.BlockSpec((mb, REL_DIM), lambda i: (i, 0)),
            pl.BlockSpec((REL_DIM, HID), lambda i: (0, 0)),
            pl.BlockSpec((1, HID), lambda i: (0, 0)),
        ],
        out_specs=pl.BlockSpec((mb, HID), lambda i: (i, 0)),
        out_shape=jax.ShapeDtypeStruct((N_REL, HID), jnp.float32),
    )(vr, w, b)


def _objmm_body(f_ref, w_ref, b_ref, lg_ref, mask_ref, probs_ref, wemb_ref,
                h_ref, pred_ref):
    lg = lg_ref[...]
    mask = mask_ref[...]
    lane = lax.broadcasted_iota(jnp.int32, lg.shape, 1)
    lp = mask * lg + (1.0 - mask) * (-1000.0)
    lp = jnp.where(lane < NUM_OBJ_CLS, lp, NEG)
    m = jnp.max(lp, axis=1, keepdims=True)
    e = jnp.exp(lp - m)
    p2 = e / jnp.sum(e, axis=1, keepdims=True)
    emb = jnp.dot(p2, wemb_ref[...], preferred_element_type=jnp.float32)
    h = jnp.dot(f_ref[...], w_ref[...], preferred_element_type=jnp.float32)
    h_ref[...] = jnp.tanh(h + b_ref[...] + emb)

    mp = mask * probs_ref[...]
    adj = jnp.where((lane >= 1) & (lane < NUM_OBJ_CLS), mp, -1.0)
    mx = jnp.max(adj, axis=1, keepdims=True)
    cand = jnp.where(adj == mx, lane, NUM_OBJ_CLS + 1)
    pred_ref[...] = jnp.min(cand, axis=1, keepdims=True)


def _objmm_call(fmaps, w, b, logits_p, mask_p, probs, wemb_p):
    mb = 200
    return pl.pallas_call(
        _objmm_body,
        grid=(N_OBJ // mb,),
        in_specs=[
            pl.BlockSpec((mb, OBJ_DIM), lambda i: (i, 0)),
            pl.BlockSpec((OBJ_DIM, HID), lambda i: (0, 0)),
            pl.BlockSpec((1, HID), lambda i: (0, 0)),
            pl.BlockSpec((mb, CP), lambda i: (i, 0)),
            pl.BlockSpec((mb, CP), lambda i: (i, 0)),
            pl.BlockSpec((mb, CP), lambda i: (i, 0)),
            pl.BlockSpec((CP, HID), lambda i: (0, 0)),
        ],
        out_specs=[
            pl.BlockSpec((mb, HID), lambda i: (i, 0)),
            pl.BlockSpec((mb, 1), lambda i: (i, 0)),
        ],
        out_shape=[
            jax.ShapeDtypeStruct((N_OBJ, HID), jnp.float32),
            jax.ShapeDtypeStruct((N_OBJ, 1), jnp.int32),
        ],
    )(fmaps, w, b, logits_p, mask_p, probs, wemb_p)


def _ggnn_body(vr0_ref, objh0_ref, subr_ref, obr_ref, subc_ref, obc_ref,
               wmsg_ref, wout_ref, bout_ref, out_ref, vr_ref, objh_ref):
    nblk = NP_REL // RB
    vr_ref[...] = vr0_ref[...]
    objh_ref[...] = objh0_ref[...]

    iota_obj_row = lax.broadcasted_iota(jnp.int32, (NP_OBJ, RB), 0)
    iota_obj_lane = lax.broadcasted_iota(jnp.int32, (RB, NP_OBJ), 1)

    for _ in range(T_STEPS):
        msg = jnp.zeros((NP_OBJ, HID), jnp.float32)
        for rb in range(nblk):
            sub_blk = subr_ref[:, rb * RB:(rb + 1) * RB]
            ob_blk = obr_ref[:, rb * RB:(rb + 1) * RB]
            oh = ((sub_blk == iota_obj_row).astype(jnp.float32)
                  + (ob_blk == iota_obj_row).astype(jnp.float32))
            vr_blk = vr_ref[rb * RB:(rb + 1) * RB, :]
            msg = msg + jnp.dot(oh, vr_blk, preferred_element_type=jnp.float32)
        nrm = jnp.sqrt(jnp.sum(msg * msg, axis=1, keepdims=True))
        msg = msg / (nrm + 1e-8)
        objh_ref[...] = jnp.tanh(
            objh_ref[...]
            + jnp.dot(msg, wmsg_ref[...], preferred_element_type=jnp.float32))
        obj_h = objh_ref[...]
        for rb in range(nblk):
            sub_blk = subc_ref[rb * RB:(rb + 1) * RB, :]
            ob_blk = obc_ref[rb * RB:(rb + 1) * RB, :]
            g = ((sub_blk == iota_obj_lane).astype(jnp.float32)
                 + (ob_blk == iota_obj_lane).astype(jnp.float32))
            gsum = jnp.dot(g, obj_h, preferred_element_type=jnp.float32)
            vr_ref[rb * RB:(rb + 1) * RB, :] = jnp.tanh(
                vr_ref[rb * RB:(rb + 1) * RB, :] + gsum)

    obj_h = objh_ref[...]
    w1 = wout_ref[0:HID, :]
    w2 = wout_ref[HID:2 * HID, :]
    w3 = wout_ref[2 * HID:3 * HID, :]
    a1 = jnp.dot(obj_h, w1, preferred_element_type=jnp.float32)
    a2 = jnp.dot(obj_h, w2, preferred_element_type=jnp.float32)
    for rb in range(nblk):
        sub_blk = subc_ref[rb * RB:(rb + 1) * RB, :]
        ob_blk = obc_ref[rb * RB:(rb + 1) * RB, :]
        gs = (sub_blk == iota_obj_lane).astype(jnp.float32)
        go = (ob_blk == iota_obj_lane).astype(jnp.float32)
        out_ref[rb * RB:(rb + 1) * RB, :] = (
            jnp.dot(gs, a1, preferred_element_type=jnp.float32)
            + jnp.dot(go, a2, preferred_element_type=jnp.float32)
            + jnp.dot(vr_ref[rb * RB:(rb + 1) * RB, :], w3,
                      preferred_element_type=jnp.float32)
            + bout_ref[...])


def _ggnn_call(vr0_p, objh0_p, sub_r, ob_r, sub_c, ob_c, wmsg, wout_p, bout_p):
    return pl.pallas_call(
        _ggnn_body,
        scratch_shapes=[
            pltpu.VMEM((NP_REL, HID), jnp.float32),
            pltpu.VMEM((NP_OBJ, HID), jnp.float32),
        ],
        out_shape=jax.ShapeDtypeStruct((NP_REL, 128), jnp.float32),
    )(vr0_p, objh0_p, sub_r, ob_r, sub_c, ob_c, wmsg, wout_p, bout_p)


def kernel(im_inds, obj_fmaps, obj_logits, rel_inds, vr, boxes_per_cls,
           W_obj, b_obj, W_rel, b_rel, W_emb, W_msg, W_out, b_out):
    f32 = jnp.float32
    nclass = NUM_OBJ_CLS - 1

    logits_p = jnp.pad(obj_logits, ((0, 0), (0, CP - NUM_OBJ_CLS)),
                       constant_values=NEG)
    probs = _probs_call(logits_p)

    bt = jnp.transpose(boxes_per_cls[:, 1:, :], (1, 2, 0))
    bt = jnp.pad(bt, ((0, 0), (0, 0), (0, NP_OBJ - N_OBJ)))
    st = jnp.transpose(probs[:, 1:NUM_OBJ_CLS])
    st = jnp.pad(st, ((0, 0), (0, NP_OBJ - N_OBJ)), constant_values=-1.0)
    rows = [bt[:, 0].reshape(nclass, 1, NP_OBJ),
            bt[:, 1].reshape(nclass, 1, NP_OBJ),
            bt[:, 2].reshape(nclass, 1, NP_OBJ),
            bt[:, 3].reshape(nclass, 1, NP_OBJ),
            st.reshape(nclass, 1, NP_OBJ)]
    cols = [bt[:, 0].reshape(nclass, NP_OBJ, 1),
            bt[:, 1].reshape(nclass, NP_OBJ, 1),
            bt[:, 2].reshape(nclass, NP_OBJ, 1),
            bt[:, 3].reshape(nclass, NP_OBJ, 1),
            st.reshape(nclass, NP_OBJ, 1)]
    keep = _nms_call(rows, cols)
    keep2 = jnp.transpose(keep[:, 0, :N_OBJ])
    mask = jnp.concatenate([jnp.zeros((N_OBJ, 1), f32), keep2], axis=1)
    mask_p = jnp.pad(mask, ((0, 0), (0, CP - NUM_OBJ_CLS)))

    vr_h0 = _relmm_call(vr, W_rel, b_rel.reshape(1, HID))
    wemb_p = jnp.pad(W_emb, ((0, CP - NUM_OBJ_CLS), (0, 0)))
    obj_h0, preds = _objmm_call(obj_fmaps, W_obj, b_obj.reshape(1, HID),
                                logits_p, mask_p, probs, wemb_p)

    sub = rel_inds[:, 1]
    ob = rel_inds[:, 2]
    sub_p = jnp.pad(sub, (0, NP_REL - N_REL), constant_values=-1)
    ob_p = jnp.pad(ob, (0, NP_REL - N_REL), constant_values=-1)
    vr0_p = jnp.pad(vr_h0, ((0, NP_REL - N_REL), (0, 0)))
    objh0_p = jnp.pad(obj_h0, ((0, NP_OBJ - N_OBJ), (0, 0)))
    wout_p = jnp.pad(W_out, ((0, 0), (0, 128 - NUM_REL_CLS)))
    bout_p = jnp.pad(b_out, (0, 128 - NUM_REL_CLS)).reshape(1, 128)
    rel_p = _ggnn_call(vr0_p, objh0_p,
                       sub_p.reshape(1, NP_REL), ob_p.reshape(1, NP_REL),
                       sub_p.reshape(NP_REL, 1), ob_p.reshape(NP_REL, 1),
                       W_msg, wout_p, bout_p)
    rel_logits = rel_p[:N_REL, :NUM_REL_CLS]

    obj_preds = preds.reshape(N_OBJ).astype(jnp.int32)
    return (obj_logits, obj_preds, rel_logits)
